# baseline (speedup 1.0000x reference)
import functools

import jax
import jax.numpy as jnp
from jax.experimental import pallas as pl
from jax.experimental.pallas import tpu as pltpu


def _stats_body(x_ref, out_ref, xbf_ref):
    xb = x_ref[...]
    xbf_ref[...] = xb.astype(jnp.bfloat16)
    s = jnp.sum(xb, axis=0)
    q = jnp.sum(xb * xb, axis=0)
    partial = jnp.stack([s, q])

    @pl.when(pl.program_id(0) == 0)
    def _init():
        out_ref[...] = partial

    @pl.when(pl.program_id(0) != 0)
    def _acc():
        out_ref[...] = out_ref[...] + partial


def _rnn_body(x_ref, stats_ref, bn_ref, Wx_ref, Wh_ref, b_ref, Wd_ref,
              bd_ref, out_ref, hall_ref, Wx2_ref, b2_ref, *, inv_m, T):
    bidx = pl.program_id(0)

    @pl.when(bidx == 0)
    def _fold_bn():
        mean = stats_ref[0:1, :] * inv_m
        var = stats_ref[1:2, :] * inv_m - mean * mean
        scale = bn_ref[0:1, :] * jax.lax.rsqrt(var + 1e-5)
        shift = bn_ref[1:2, :] - mean * scale
        Wx2_ref[...] = (Wx_ref[...] * jnp.transpose(scale)).astype(
            jnp.bfloat16)
        b2_ref[...] = b_ref[...] + jnp.dot(
            shift, Wx_ref[...], preferred_element_type=jnp.float32)

    nb = hall_ref.shape[0] // T
    b2 = b2_ref[...]
    p = jnp.dot(x_ref[...], Wx2_ref[...],
                preferred_element_type=jnp.float32) + b2
    chains = 4
    c = nb // chains
    hs = [None] * chains
    for t in range(T):
        base = t * nb
        for k in range(chains):
            lo = base + k * c
            pre = p[lo:lo + c]
            if hs[k] is not None:
                pre = pre + jnp.dot(hs[k], Wh_ref[...],
                                    preferred_element_type=jnp.float32)
            hs[k] = jnp.tanh(pre).astype(jnp.bfloat16)
            hall_ref[lo:lo + c] = hs[k]

    res = jnp.dot(hall_ref[...], Wd_ref[...],
                  preferred_element_type=jnp.float32)
    out_ref[...] = res + bd_ref[...]


def kernel(x, bn_gamma, bn_beta, Wx, Wh, b, Wd, bd):
    B, T, N, F = x.shape
    H = Wh.shape[0]
    O = Wd.shape[1]
    M = B * T * N

    rows = N
    for cand in (16000, 8000, 4000, 2000):
        if M % cand == 0:
            rows = cand
            break
    x2 = x.reshape(M, F)
    stats, xbf = pl.pallas_call(
        _stats_body,
        grid=(M // rows,),
        in_specs=[pl.BlockSpec((rows, F), lambda i: (i, 0))],
        out_specs=[
            pl.BlockSpec((2, F), lambda i: (0, 0)),
            pl.BlockSpec((rows, F), lambda i: (i, 0)),
        ],
        out_shape=[
            jax.ShapeDtypeStruct((2, F), jnp.float32),
            jax.ShapeDtypeStruct((M, F), jnp.bfloat16),
        ],
    )(x2)

    tn = T * N
    bn = jnp.stack([bn_gamma, bn_beta])
    full = lambda shape: pl.BlockSpec(shape, lambda bi: (0, 0))
    out = pl.pallas_call(
        functools.partial(_rnn_body, inv_m=1.0 / M, T=T),
        grid=(B,),
        in_specs=[
            pl.BlockSpec((tn, F), lambda bi: (bi, 0)),
            full((2, F)),
            full((2, F)),
            full((F, H)),
            full((H, H)),
            full((1, H)),
            full((H, O)),
            full((1, O)),
        ],
        out_specs=pl.BlockSpec((tn, O), lambda bi: (bi, 0)),
        out_shape=jax.ShapeDtypeStruct((M, O), jnp.float32),
        scratch_shapes=[
            pltpu.VMEM((tn, H), jnp.bfloat16),
            pltpu.VMEM((F, H), jnp.bfloat16),
            pltpu.VMEM((1, H), jnp.float32),
        ],
        compiler_params=pltpu.CompilerParams(
            vmem_limit_bytes=100 * 1024 * 1024),
    )(xbf, stats, bn, Wx, Wh.astype(jnp.bfloat16), b.reshape(1, H),
      Wd.astype(jnp.bfloat16), bd.reshape(1, O))
    return out.reshape(B, T, N, O)


# speedup vs baseline: 1.0659x; 1.0659x over previous
import functools

import jax
import jax.numpy as jnp
from jax.experimental import pallas as pl
from jax.experimental.pallas import tpu as pltpu


def _stats_body(x_ref, out_ref, xbf_ref):
    xb = x_ref[...]
    xbf_ref[...] = xb.astype(jnp.bfloat16)
    s = jnp.sum(xb, axis=0)
    q = jnp.sum(xb * xb, axis=0)
    partial = jnp.stack([s, q])

    @pl.when(pl.program_id(0) == 0)
    def _init():
        out_ref[...] = partial

    @pl.when(pl.program_id(0) != 0)
    def _acc():
        out_ref[...] = out_ref[...] + partial


def _rnn_body(x_ref, stats_ref, bn_ref, Wx_ref, Wh_ref, b_ref, Wd_ref,
              bd_ref, out_ref, hall_ref, Wx2_ref, b2_ref, *, inv_m, T):
    bidx = pl.program_id(0)

    @pl.when(bidx == 0)
    def _fold_bn():
        mean = stats_ref[0:1, :] * inv_m
        var = stats_ref[1:2, :] * inv_m - mean * mean
        scale = bn_ref[0:1, :] * jax.lax.rsqrt(var + 1e-5)
        shift = bn_ref[1:2, :] - mean * scale
        Wx2_ref[...] = (Wx_ref[...] * jnp.transpose(scale)).astype(
            jnp.bfloat16)
        b2_ref[...] = b_ref[...] + jnp.dot(
            shift, Wx_ref[...], preferred_element_type=jnp.float32)

    nb = hall_ref.shape[0] // T
    b2 = b2_ref[...]
    p = jnp.dot(x_ref[...], Wx2_ref[...],
                preferred_element_type=jnp.float32) + b2
    chains = 2
    c = nb // chains
    hs = [None] * chains
    for t in range(T):
        base = t * nb
        for k in range(chains):
            lo = base + k * c
            pre = p[lo:lo + c]
            if hs[k] is not None:
                pre = pre + jnp.dot(hs[k], Wh_ref[...],
                                    preferred_element_type=jnp.float32)
            hs[k] = jnp.tanh(pre).astype(jnp.bfloat16)
            hall_ref[lo:lo + c] = hs[k]

    res = jnp.dot(hall_ref[...], Wd_ref[...],
                  preferred_element_type=jnp.float32)
    out_ref[...] = res + bd_ref[...]


def kernel(x, bn_gamma, bn_beta, Wx, Wh, b, Wd, bd):
    B, T, N, F = x.shape
    H = Wh.shape[0]
    O = Wd.shape[1]
    M = B * T * N

    rows = N
    for cand in (16000, 8000, 4000, 2000):
        if M % cand == 0:
            rows = cand
            break
    x2 = x.reshape(M, F)
    stats, xbf = pl.pallas_call(
        _stats_body,
        grid=(M // rows,),
        in_specs=[pl.BlockSpec((rows, F), lambda i: (i, 0))],
        out_specs=[
            pl.BlockSpec((2, F), lambda i: (0, 0)),
            pl.BlockSpec((rows, F), lambda i: (i, 0)),
        ],
        out_shape=[
            jax.ShapeDtypeStruct((2, F), jnp.float32),
            jax.ShapeDtypeStruct((M, F), jnp.bfloat16),
        ],
    )(x2)

    tn = T * N
    bn = jnp.stack([bn_gamma, bn_beta])
    full = lambda shape: pl.BlockSpec(shape, lambda bi: (0, 0))
    out = pl.pallas_call(
        functools.partial(_rnn_body, inv_m=1.0 / M, T=T),
        grid=(B,),
        in_specs=[
            pl.BlockSpec((tn, F), lambda bi: (bi, 0)),
            full((2, F)),
            full((2, F)),
            full((F, H)),
            full((H, H)),
            full((1, H)),
            full((H, O)),
            full((1, O)),
        ],
        out_specs=pl.BlockSpec((tn, O), lambda bi: (bi, 0)),
        out_shape=jax.ShapeDtypeStruct((M, O), jnp.float32),
        scratch_shapes=[
            pltpu.VMEM((tn, H), jnp.bfloat16),
            pltpu.VMEM((F, H), jnp.bfloat16),
            pltpu.VMEM((1, H), jnp.float32),
        ],
        compiler_params=pltpu.CompilerParams(
            vmem_limit_bytes=100 * 1024 * 1024),
    )(xbf, stats, bn, Wx, Wh.astype(jnp.bfloat16), b.reshape(1, H),
      Wd.astype(jnp.bfloat16), bd.reshape(1, O))
    return out.reshape(B, T, N, O)
